# baseline (speedup 1.0000x reference)
import functools

import jax
import jax.numpy as jnp
from jax import lax
from jax.experimental import pallas as pl
from jax.experimental.pallas import tpu as pltpu
from jax.experimental.pallas import tpu_sc as plsc

Z_DIM = 101
LATENT_DIM = 92
N_ATOMS = 100000

L = 16
ROWS_PER_CHUNK = 125
N_CHUNKS = N_ATOMS // ROWS_PER_CHUNK
N_WORKERS = 32
CHUNKS_PER_WORKER = N_CHUNKS // N_WORKERS
ROWS_PER_WORKER = ROWS_PER_CHUNK * CHUNKS_PER_WORKER
IDX_PAD = 3136
N_GROUPS = 8
TAIL_LANES = ROWS_PER_CHUNK - (N_GROUPS - 1) * L


def _make_sc_gather():
    mesh = plsc.VectorSubcoreMesh(core_axis_name="c", subcore_axis_name="s")
    nc = mesh.num_cores

    @functools.partial(
        pl.kernel,
        out_type=jax.ShapeDtypeStruct((N_CHUNKS, ROWS_PER_CHUNK, LATENT_DIM),
                                      jnp.float32),
        mesh=mesh,
        scratch_types=[
            pltpu.VMEM((Z_DIM * LATENT_DIM,), jnp.float32),
            pltpu.VMEM((1, IDX_PAD), jnp.int32),
            pltpu.VMEM((ROWS_PER_CHUNK, LATENT_DIM), jnp.float32),
        ],
        compiler_params=pltpu.CompilerParams(use_tc_tiling_on_sc=False,
                                             needs_layout_passes=False),
    )
    def gather_kernel(z_hbm, tab_hbm, out_hbm, tab_v, idx_v, stag_v):
        wid = lax.axis_index("s") * nc + lax.axis_index("c")
        base = wid * CHUNKS_PER_WORKER
        pltpu.sync_copy(tab_hbm, tab_v)
        pltpu.sync_copy(z_hbm.at[wid], idx_v)

        lanes = lax.iota(jnp.int32, L)
        zeros = jnp.zeros((L,), jnp.int32)
        tail_mask = lanes < TAIL_LANES

        def chunk(c, carry):
            for g in range(N_GROUPS):
                pos = c * ROWS_PER_CHUNK + g * L + lanes
                vidx = plsc.load_gather(idx_v, [zeros, pos])
                vbase = vidx * LATENT_DIM
                vrow = g * L + lanes
                msk = None if g < N_GROUPS - 1 else tail_mask
                for j in range(LATENT_DIM):
                    vals = plsc.load_gather(tab_v, [vbase + j])
                    vcol = zeros + j
                    plsc.store_scatter(stag_v, [vrow, vcol], vals, mask=msk)
            pltpu.sync_copy(stag_v, out_hbm.at[base + c])
            return carry

        lax.fori_loop(0, CHUNKS_PER_WORKER, chunk, 0)

    return gather_kernel


_sc_gather = _make_sc_gather()


@jax.jit
def kernel(z, features):
    zw = z.reshape(N_WORKERS, ROWS_PER_WORKER)
    zw = jnp.pad(zw, ((0, 0), (0, IDX_PAD - ROWS_PER_WORKER)))
    z3 = zw.reshape(N_WORKERS, 1, IDX_PAD)
    tab = features.reshape(Z_DIM * LATENT_DIM)
    out = _sc_gather(z3, tab)
    return out.reshape(N_ATOMS, LATENT_DIM)


# speedup vs baseline: 1.0141x; 1.0141x over previous
import functools

import jax
import jax.numpy as jnp
from jax import lax
from jax.experimental import pallas as pl
from jax.experimental.pallas import tpu as pltpu
from jax.experimental.pallas import tpu_sc as plsc

Z_DIM = 101
LATENT_DIM = 92
N_ATOMS = 100000

L = 16
ROWS_PER_CHUNK = 125
N_CHUNKS = N_ATOMS // ROWS_PER_CHUNK
N_WORKERS = 32
CHUNKS_PER_WORKER = N_CHUNKS // N_WORKERS
ROWS_PER_WORKER = ROWS_PER_CHUNK * CHUNKS_PER_WORKER
IDX_PAD = 3136
N_GROUPS = 8
TAIL_LANES = ROWS_PER_CHUNK - (N_GROUPS - 1) * L
JU = 4


def _make_sc_gather():
    mesh = plsc.VectorSubcoreMesh(core_axis_name="c", subcore_axis_name="s")
    nc = mesh.num_cores

    @functools.partial(
        pl.kernel,
        out_type=jax.ShapeDtypeStruct((N_CHUNKS, ROWS_PER_CHUNK, LATENT_DIM),
                                      jnp.float32),
        mesh=mesh,
        scratch_types=[
            pltpu.VMEM((Z_DIM * LATENT_DIM,), jnp.float32),
            pltpu.VMEM((1, IDX_PAD), jnp.int32),
            pltpu.VMEM((ROWS_PER_CHUNK, LATENT_DIM), jnp.float32),
        ],
        compiler_params=pltpu.CompilerParams(use_tc_tiling_on_sc=False,
                                             needs_layout_passes=False),
    )
    def gather_kernel(z_hbm, tab_hbm, out_hbm, tab_v, idx_v, stag_v):
        wid = lax.axis_index("s") * nc + lax.axis_index("c")
        base = wid * CHUNKS_PER_WORKER
        pltpu.sync_copy(tab_hbm, tab_v)
        pltpu.sync_copy(z_hbm.at[wid], idx_v)

        lanes = lax.iota(jnp.int32, L)
        zeros = jnp.zeros((L,), jnp.int32)
        tail_mask = lanes < TAIL_LANES

        def chunk(c, carry):
            vbases = []
            for g in range(N_GROUPS):
                pos = c * ROWS_PER_CHUNK + g * L + lanes
                vidx = plsc.load_gather(idx_v, [zeros, pos])
                vbases.append(vidx * LATENT_DIM)

            def col(jb, cols):
                j0 = jb * JU
                for u in range(JU):
                    vcol = cols[u]
                    for g in range(N_GROUPS):
                        vals = plsc.load_gather(tab_v, [vbases[g] + (j0 + u)])
                        vrow = g * L + lanes
                        msk = None if g < N_GROUPS - 1 else tail_mask
                        plsc.store_scatter(stag_v, [vrow, vcol], vals,
                                           mask=msk)
                return tuple(v + JU for v in cols)

            cols0 = tuple(zeros + u for u in range(JU))
            lax.fori_loop(0, LATENT_DIM // JU, col, cols0)
            pltpu.sync_copy(stag_v, out_hbm.at[base + c])
            return carry

        lax.fori_loop(0, CHUNKS_PER_WORKER, chunk, 0)

    return gather_kernel


_sc_gather = _make_sc_gather()


@jax.jit
def kernel(z, features):
    zw = z.reshape(N_WORKERS, ROWS_PER_WORKER)
    zw = jnp.pad(zw, ((0, 0), (0, IDX_PAD - ROWS_PER_WORKER)))
    z3 = zw.reshape(N_WORKERS, 1, IDX_PAD)
    tab = features.reshape(Z_DIM * LATENT_DIM)
    out = _sc_gather(z3, tab)
    return out.reshape(N_ATOMS, LATENT_DIM)


# speedup vs baseline: 1.5292x; 1.5079x over previous
import functools

import jax
import jax.numpy as jnp
from jax import lax
from jax.experimental import pallas as pl
from jax.experimental.pallas import tpu as pltpu
from jax.experimental.pallas import tpu_sc as plsc

Z_DIM = 101
LATENT_DIM = 92
N_ATOMS = 100000

L = 16
ROWS_PER_CHUNK = 125
N_CHUNKS = N_ATOMS // ROWS_PER_CHUNK
N_WORKERS = 32
CHUNKS_PER_WORKER = N_CHUNKS // N_WORKERS
ROWS_PER_WORKER = ROWS_PER_CHUNK * CHUNKS_PER_WORKER
IDX_PAD = 3136
N_COLG = 6
RU = 5


def _make_sc_gather():
    mesh = plsc.VectorSubcoreMesh(core_axis_name="c", subcore_axis_name="s")
    nc = mesh.num_cores

    @functools.partial(
        pl.kernel,
        out_type=jax.ShapeDtypeStruct((N_CHUNKS, ROWS_PER_CHUNK, LATENT_DIM),
                                      jnp.float32),
        mesh=mesh,
        scratch_types=[
            pltpu.VMEM((Z_DIM, LATENT_DIM), jnp.float32),
            pltpu.VMEM((1, IDX_PAD), jnp.int32),
            pltpu.VMEM((ROWS_PER_CHUNK, LATENT_DIM), jnp.float32),
        ],
        compiler_params=pltpu.CompilerParams(use_tc_tiling_on_sc=False,
                                             needs_layout_passes=False),
    )
    def gather_kernel(z_hbm, tab_hbm, out_hbm, tab_v, idx_v, stag_v):
        wid = lax.axis_index("s") * nc + lax.axis_index("c")
        base = wid * CHUNKS_PER_WORKER
        pltpu.sync_copy(tab_hbm, tab_v)
        pltpu.sync_copy(z_hbm.at[wid], idx_v)

        lanes = lax.iota(jnp.int32, L)
        zeros = jnp.zeros((L,), jnp.int32)
        tail_mask = lanes < LATENT_DIM - (N_COLG - 1) * L
        colvecs = [g * L + lanes for g in range(N_COLG)]

        def chunk(c, carry):
            def rowblk(rb, carry2):
                for u in range(RU):
                    r = rb * RU + u
                    p = c * ROWS_PER_CHUNK + r
                    vidx = plsc.load_gather(idx_v, [zeros, zeros + p])
                    for g in range(N_COLG - 1):
                        vals = plsc.load_gather(tab_v, [vidx, colvecs[g]])
                        stag_v[r, pl.ds(g * L, L)] = vals
                    vals = plsc.load_gather(tab_v, [vidx, colvecs[-1]],
                                            mask=tail_mask)
                    plsc.store_scatter(stag_v, [zeros + r, colvecs[-1]],
                                       vals, mask=tail_mask)
                return carry2

            lax.fori_loop(0, ROWS_PER_CHUNK // RU, rowblk, 0)
            pltpu.sync_copy(stag_v, out_hbm.at[base + c])
            return carry

        lax.fori_loop(0, CHUNKS_PER_WORKER, chunk, 0)

    return gather_kernel


_sc_gather = _make_sc_gather()


@jax.jit
def kernel(z, features):
    zw = z.reshape(N_WORKERS, ROWS_PER_WORKER)
    zw = jnp.pad(zw, ((0, 0), (0, IDX_PAD - ROWS_PER_WORKER)))
    z3 = zw.reshape(N_WORKERS, 1, IDX_PAD)
    out = _sc_gather(z3, features)
    return out.reshape(N_ATOMS, LATENT_DIM)


# speedup vs baseline: 2.2457x; 1.4686x over previous
import functools

import jax
import jax.numpy as jnp
from jax import lax
from jax.experimental import pallas as pl
from jax.experimental.pallas import tpu as pltpu
from jax.experimental.pallas import tpu_sc as plsc

Z_DIM = 101
LATENT_DIM = 92
N_ATOMS = 100000

L = 16
ROWS_PER_CHUNK = 125
N_CHUNKS = N_ATOMS // ROWS_PER_CHUNK
N_WORKERS = 32
CHUNKS_PER_WORKER = N_CHUNKS // N_WORKERS
ROWS_PER_WORKER = ROWS_PER_CHUNK * CHUNKS_PER_WORKER
IDX_PAD = 3136
N_COLG = 6
RU = 5


def _make_sc_gather():
    mesh = plsc.VectorSubcoreMesh(core_axis_name="c", subcore_axis_name="s")
    nc = mesh.num_cores

    @functools.partial(
        pl.kernel,
        out_type=jax.ShapeDtypeStruct((N_CHUNKS, ROWS_PER_CHUNK, LATENT_DIM),
                                      jnp.float32),
        mesh=mesh,
        scratch_types=[
            pltpu.VMEM((Z_DIM, LATENT_DIM), jnp.float32),
            pltpu.VMEM((1, IDX_PAD), jnp.int32),
            pltpu.VMEM((ROWS_PER_CHUNK, LATENT_DIM), jnp.float32),
        ],
        compiler_params=pltpu.CompilerParams(use_tc_tiling_on_sc=False,
                                             needs_layout_passes=False),
    )
    def gather_kernel(z_hbm, tab_hbm, out_hbm, tab_v, idx_v, stag_v):
        wid = lax.axis_index("s") * nc + lax.axis_index("c")
        base = wid * CHUNKS_PER_WORKER
        pltpu.sync_copy(tab_hbm, tab_v)
        pltpu.sync_copy(z_hbm.at[wid], idx_v)

        lanes = lax.iota(jnp.int32, L)
        zeros = jnp.zeros((L,), jnp.int32)
        tail_mask = lanes < LATENT_DIM - (N_COLG - 1) * L
        colvecs = [g * L + lanes for g in range(N_COLG)]

        def chunk(c, carry):
            @plsc.parallel_loop(0, ROWS_PER_CHUNK, unroll=RU)
            def rowbody(r):
                p = c * ROWS_PER_CHUNK + r
                vidx = plsc.load_gather(idx_v, [zeros, zeros + p])
                for g in range(N_COLG - 1):
                    vals = plsc.load_gather(tab_v, [vidx, colvecs[g]])
                    stag_v[r, pl.ds(g * L, L)] = vals
                vals = plsc.load_gather(tab_v, [vidx, colvecs[-1]],
                                        mask=tail_mask)
                plsc.store_scatter(stag_v, [zeros + r, colvecs[-1]],
                                   vals, mask=tail_mask)

            pltpu.sync_copy(stag_v, out_hbm.at[base + c])
            return carry

        lax.fori_loop(0, CHUNKS_PER_WORKER, chunk, 0)

    return gather_kernel


_sc_gather = _make_sc_gather()


@jax.jit
def kernel(z, features):
    zw = z.reshape(N_WORKERS, ROWS_PER_WORKER)
    zw = jnp.pad(zw, ((0, 0), (0, IDX_PAD - ROWS_PER_WORKER)))
    z3 = zw.reshape(N_WORKERS, 1, IDX_PAD)
    out = _sc_gather(z3, features)
    return out.reshape(N_ATOMS, LATENT_DIM)


# speedup vs baseline: 2.2484x; 1.0012x over previous
import functools

import jax
import jax.numpy as jnp
from jax import lax
from jax.experimental import pallas as pl
from jax.experimental.pallas import tpu as pltpu
from jax.experimental.pallas import tpu_sc as plsc

Z_DIM = 101
LATENT_DIM = 92
N_ATOMS = 100000

L = 16
ROWS_PER_CHUNK = 125
N_CHUNKS = N_ATOMS // ROWS_PER_CHUNK
N_WORKERS = 32
CHUNKS_PER_WORKER = N_CHUNKS // N_WORKERS
ROWS_PER_WORKER = ROWS_PER_CHUNK * CHUNKS_PER_WORKER
IDX_PAD = 3136
N_COLG = 6
RU = 5
TAB_PAD = Z_DIM * LATENT_DIM + 20


def _make_sc_gather():
    mesh = plsc.VectorSubcoreMesh(core_axis_name="c", subcore_axis_name="s")
    nc = mesh.num_cores

    @functools.partial(
        pl.kernel,
        out_type=jax.ShapeDtypeStruct((N_CHUNKS, ROWS_PER_CHUNK, LATENT_DIM),
                                      jnp.float32),
        mesh=mesh,
        scratch_types=[
            pltpu.VMEM((TAB_PAD,), jnp.float32),
            pltpu.VMEM((1, IDX_PAD), jnp.int32),
            pltpu.VMEM((ROWS_PER_CHUNK, LATENT_DIM), jnp.float32),
        ],
        compiler_params=pltpu.CompilerParams(use_tc_tiling_on_sc=False,
                                             needs_layout_passes=False),
    )
    def gather_kernel(z_hbm, tab_hbm, out_hbm, tab_v, idx_v, stag_v):
        wid = lax.axis_index("s") * nc + lax.axis_index("c")
        base = wid * CHUNKS_PER_WORKER
        pltpu.sync_copy(tab_hbm, tab_v)
        pltpu.sync_copy(z_hbm.at[wid], idx_v)

        lanes = lax.iota(jnp.int32, L)
        zeros = jnp.zeros((L,), jnp.int32)
        tail_mask = lanes < LATENT_DIM - (N_COLG - 1) * L
        colvecs = [g * L + lanes for g in range(N_COLG)]

        def chunk(c, carry):
            @plsc.parallel_loop(0, ROWS_PER_CHUNK, step=RU, unroll=5)
            def rowblk(r0):
                vidx = idx_v[0, pl.ds(c * ROWS_PER_CHUNK + r0, L)]
                for u in range(RU):
                    r = r0 + u
                    s = vidx[u] * LATENT_DIM
                    for g in range(N_COLG - 1):
                        stag_v[r, pl.ds(g * L, L)] = \
                            tab_v[pl.ds(s + g * L, L)]
                    vals = tab_v[pl.ds(s + (N_COLG - 1) * L, L)]
                    plsc.store_scatter(stag_v, [zeros + r, colvecs[-1]],
                                       vals, mask=tail_mask)

            pltpu.sync_copy(stag_v, out_hbm.at[base + c])
            return carry

        lax.fori_loop(0, CHUNKS_PER_WORKER, chunk, 0)

    return gather_kernel


_sc_gather = _make_sc_gather()


@jax.jit
def kernel(z, features):
    zw = z.reshape(N_WORKERS, ROWS_PER_WORKER)
    zw = jnp.pad(zw, ((0, 0), (0, IDX_PAD - ROWS_PER_WORKER)))
    z3 = zw.reshape(N_WORKERS, 1, IDX_PAD)
    tab = jnp.pad(features.reshape(Z_DIM * LATENT_DIM),
                  (0, TAB_PAD - Z_DIM * LATENT_DIM))
    out = _sc_gather(z3, tab)
    return out.reshape(N_ATOMS, LATENT_DIM)
